# baseline (speedup 1.0000x reference)
import functools

import numpy as np

import jax
import jax.numpy as jnp
from jax import lax
from jax.experimental import pallas as pl
from jax.experimental.pallas import tpu as pltpu
from jax.experimental.pallas import tpu_sc as plsc

B = 16384
D = 8
NCOLS = 37
NC, NS, L = 2, 16, 16
NW = NC * NS
BPW = B // NW
CHUNK = 128
NCHUNK = BPW // CHUNK
GROUP = L
NGROUP = BPW // GROUP
GWORDS = GROUP * NCOLS

_mesh = plsc.VectorSubcoreMesh(
    core_axis_name="c", subcore_axis_name="s", num_cores=NC, num_subcores=NS
)


@functools.partial(
    pl.kernel,
    out_type=jax.ShapeDtypeStruct((B * NCOLS,), jnp.float32),
    mesh=_mesh,
    compiler_params=pltpu.CompilerParams(
        needs_layout_passes=False, use_tc_tiling_on_sc=False
    ),
    scratch_types=[
        pltpu.VMEM((BPW,), jnp.int32),
        pltpu.VMEM((BPW, D), jnp.float32),
        pltpu.VMEM((BPW * 7,), jnp.float32),
        pltpu.VMEM((BPW * 21,), jnp.float32),
        pltpu.VMEM((BPW,), jnp.int32),
        pltpu.VMEM((BPW * NCOLS,), jnp.float32),
        pltpu.SemaphoreType.DMA,
    ],
)
def _encode(gender_hbm, age_hbm, occ_hbm, idx_hbm, table_hbm, out_hbm,
            idx_v, z_v, a_v, o_v, g_v, s_v, sem):
    wid = lax.axis_index("s") * NC + lax.axis_index("c")
    base = wid * BPW

    pltpu.sync_copy(idx_hbm.at[pl.ds(base, BPW)], idx_v)
    copies = []
    for j in range(NCHUNK):
        sl = pl.ds(j * CHUNK, CHUNK)
        copies.append(
            pltpu.async_copy(table_hbm.at[idx_v.at[sl]], z_v.at[sl], sem)
        )

    pltpu.sync_copy(age_hbm.at[pl.ds(base * 7, BPW * 7)], a_v)
    pltpu.sync_copy(occ_hbm.at[pl.ds(base * 21, BPW * 21)], o_v)
    pltpu.sync_copy(gender_hbm.at[pl.ds(base, BPW)], g_v)
    for c in copies:
        c.wait()

    lane = lax.iota(jnp.int32, L)

    def dsts(nvec, width, colbase):
        out = []
        for k in range(nvec):
            j = lane + k * L
            r = lax.div(j, jnp.int32(width))
            c = lax.rem(j, jnp.int32(width))
            out.append(r * NCOLS + (colbase + c))
        return out

    gender_dst = lane * NCOLS
    age_dst = dsts(7, 7, 1)
    occ_dst = dsts(21, 21, 8)
    z_dst = dsts(8, 8, 29)
    z_src_row = [lax.div(lane + k * L, jnp.int32(8)) for k in range(8)]
    z_src_col = [lax.rem(lane + k * L, jnp.int32(8)) for k in range(8)]

    def group_body(g, carry):
        goff = g * GWORDS
        grow = g * GROUP
        gvals = g_v[pl.ds(grow, L)].astype(jnp.float32)
        plsc.store_scatter(s_v, [gender_dst + goff], gvals)
        for k in range(7):
            vals = a_v[pl.ds(g * 112 + k * L, L)]
            plsc.store_scatter(s_v, [age_dst[k] + goff], vals)
        for k in range(21):
            vals = o_v[pl.ds(g * 336 + k * L, L)]
            plsc.store_scatter(s_v, [occ_dst[k] + goff], vals)
        for k in range(8):
            vals = plsc.load_gather(z_v, [z_src_row[k] + grow, z_src_col[k]])
            plsc.store_scatter(s_v, [z_dst[k] + goff], vals)
        return carry

    lax.fori_loop(0, NGROUP, group_body, 0)

    pltpu.sync_copy(s_v, out_hbm.at[pl.ds(base * NCOLS, BPW * NCOLS)])


def kernel(gender, age, occupation, zipcode_bucket, zipcode_table):
    flat = _encode(
        gender.astype(jnp.int32),
        age.reshape(-1),
        occupation.reshape(-1),
        zipcode_bucket.astype(jnp.int32),
        zipcode_table,
    )
    return flat.reshape(B, NCOLS)
